# baseline (device time: 19623 ns/iter reference)
import jax
import jax.numpy as jnp
from jax import lax
from jax.experimental import pallas as pl
from jax.experimental.pallas import tpu as pltpu

N_DEV = 4
CHUNK = 512


def kernel(x, dy, gamma):
    m, d = x.shape
    num_chunks = m // CHUNK

    def body(x_ref, dy_ref, out_ref, acc_ref, comm_ref, send_sems, recv_sems):
        i = pl.program_id(0)
        my_pos = lax.axis_index("i")
        barrier_sem = pltpu.get_barrier_semaphore()

        @pl.when(i == 0)
        def _():
            acc_ref[...] = jnp.zeros_like(acc_ref)
            for k in range(1, N_DEV):
                peer = lax.rem(my_pos + k, N_DEV)
                pl.semaphore_signal(
                    barrier_sem,
                    inc=1,
                    device_id=(peer,),
                    device_id_type=pl.DeviceIdType.MESH,
                )

        xv = x_ref[...]
        dyv = dy_ref[...]
        s1 = jnp.sum(xv, axis=1, keepdims=True)
        s2 = jnp.sum(xv * xv, axis=1, keepdims=True)
        mu = s1 * (1.0 / d)
        var = s2 * (1.0 / d) - mu * mu
        rstd = lax.rsqrt(var + 1e-5)
        xhat = xv * rstd - mu * rstd
        dgamma = jnp.sum(xhat * dyv, axis=0, keepdims=True)
        dbeta = jnp.sum(dyv, axis=0, keepdims=True)
        acc_ref[...] += jnp.concatenate([dgamma, dbeta], axis=0)

        @pl.when(i == num_chunks - 1)
        def _():
            comm_ref[N_DEV - 1] = acc_ref[...]
            pl.semaphore_wait(barrier_sem, N_DEV - 1)

            rdmas = []
            for k in range(1, N_DEV):
                peer = lax.rem(my_pos + k, N_DEV)
                rdma = pltpu.make_async_remote_copy(
                    src_ref=comm_ref.at[N_DEV - 1],
                    dst_ref=comm_ref.at[k - 1],
                    send_sem=send_sems.at[k - 1],
                    recv_sem=recv_sems.at[k - 1],
                    device_id=(peer,),
                    device_id_type=pl.DeviceIdType.MESH,
                )
                rdma.start()
                rdmas.append(rdma)

            total = acc_ref[...]
            for k, rdma in enumerate(rdmas, start=1):
                rdma.wait_recv()
                total = total + comm_ref[k - 1]
            for rdma in rdmas:
                rdma.wait_send()

            out_ref[...] = total

    return pl.pallas_call(
        body,
        grid=(num_chunks,),
        out_shape=jax.ShapeDtypeStruct((2, d), jnp.float32),
        in_specs=[
            pl.BlockSpec((CHUNK, d), lambda i: (i, 0)),
            pl.BlockSpec((CHUNK, d), lambda i: (i, 0)),
        ],
        out_specs=pl.BlockSpec((2, d), lambda i: (0, 0)),
        scratch_shapes=[
            pltpu.VMEM((2, d), jnp.float32),
            pltpu.VMEM((N_DEV, 2, d), jnp.float32),
            pltpu.SemaphoreType.DMA((N_DEV - 1,)),
            pltpu.SemaphoreType.DMA((N_DEV - 1,)),
        ],
        compiler_params=pltpu.CompilerParams(
            collective_id=0,
            dimension_semantics=("arbitrary",),
        ),
    )(x, dy)
